# Initial kernel scaffold; baseline (speedup 1.0000x reference)
import jax
import jax.numpy as jnp
from jax.experimental import pallas as pl


def kernel(features, neighbor_idx, neighbor_mask, W1, gamma1, beta1, W2, gamma2, beta2):
    raise NotImplementedError("write your pallas kernel here")


# speedup vs baseline: 17.0353x; 17.0353x over previous
import functools

import jax
import jax.numpy as jnp
from jax import lax
from jax.experimental import pallas as pl
from jax.experimental.pallas import tpu as pltpu
from jax.experimental.pallas import tpu_sc as plsc

N = 50000
C = 64
K = 27
KC = K * C
NW = 32
G = 4
GPW = 392
RPW = G * GPW
N_PAD = NW * RPW
MM_BLK = 512
FIN_BLK = 400


def _mm_kernel(x_ref, w_ref, o_ref):
    o_ref[...] = jnp.dot(x_ref[...], w_ref[...],
                         preferred_element_type=jnp.float32)


def _matmul(x, w):
    return pl.pallas_call(
        _mm_kernel,
        grid=(N_PAD // MM_BLK,),
        in_specs=[pl.BlockSpec((MM_BLK, C), lambda i: (i, 0)),
                  pl.BlockSpec((C, KC), lambda i: (0, 0))],
        out_specs=pl.BlockSpec((MM_BLK, KC), lambda i: (i, 0)),
        out_shape=jax.ShapeDtypeStruct((N_PAD, KC), jnp.float32),
    )(x, w)


def _norm_mm_kernel(x_ref, a_ref, b_ref, w_ref, o_ref):
    i = pl.program_id(0)
    x = x_ref[...]
    xn = x * a_ref[0:1, :] + b_ref[0:1, :]
    xn = jnp.where(xn >= 0, xn, 0.05 * xn)
    rows = i * MM_BLK + lax.broadcasted_iota(jnp.int32, (MM_BLK, 1), 0)
    xn = jnp.where(rows < N, xn, 0.0)
    o_ref[...] = jnp.dot(xn, w_ref[...], preferred_element_type=jnp.float32)


def _norm_matmul(x, a, b, w):
    return pl.pallas_call(
        _norm_mm_kernel,
        grid=(N_PAD // MM_BLK,),
        in_specs=[pl.BlockSpec((MM_BLK, C), lambda i: (i, 0)),
                  pl.BlockSpec((8, C), lambda i: (0, 0)),
                  pl.BlockSpec((8, C), lambda i: (0, 0)),
                  pl.BlockSpec((C, KC), lambda i: (0, 0))],
        out_specs=pl.BlockSpec((MM_BLK, KC), lambda i: (i, 0)),
        out_shape=jax.ShapeDtypeStruct((N_PAD, KC), jnp.float32),
    )(x, a, b, w)


def _stats_kernel(x_ref, s_ref, q_ref):
    i = pl.program_id(0)
    x = x_ref[...]
    s = jnp.broadcast_to(jnp.sum(x, axis=0, keepdims=True), (8, C))
    q = jnp.broadcast_to(jnp.sum(x * x, axis=0, keepdims=True), (8, C))

    @pl.when(i == 0)
    def _():
        s_ref[...] = s
        q_ref[...] = q

    @pl.when(i > 0)
    def _():
        s_ref[...] += s
        q_ref[...] += q


def _stats(x):
    return pl.pallas_call(
        _stats_kernel,
        grid=(N_PAD // MM_BLK,),
        in_specs=[pl.BlockSpec((MM_BLK, C), lambda i: (i, 0))],
        out_specs=[pl.BlockSpec((8, C), lambda i: (0, 0)),
                   pl.BlockSpec((8, C), lambda i: (0, 0))],
        out_shape=[jax.ShapeDtypeStruct((8, C), jnp.float32),
                   jax.ShapeDtypeStruct((8, C), jnp.float32)],
    )(x)


def _final_kernel(x_ref, a_ref, b_ref, o_ref):
    xn = x_ref[...] * a_ref[0:1, :] + b_ref[0:1, :]
    o_ref[...] = jnp.where(xn >= 0, xn, 0.05 * xn)


def _finalize(x, a, b):
    return pl.pallas_call(
        _final_kernel,
        grid=(N // FIN_BLK,),
        in_specs=[pl.BlockSpec((FIN_BLK, C), lambda i: (i, 0)),
                  pl.BlockSpec((8, C), lambda i: (0, 0)),
                  pl.BlockSpec((8, C), lambda i: (0, 0))],
        out_specs=pl.BlockSpec((FIN_BLK, C), lambda i: (i, 0)),
        out_shape=jax.ShapeDtypeStruct((N, C), jnp.float32),
    )(x, a, b)


@functools.partial(
    pl.kernel,
    mesh=plsc.VectorSubcoreMesh(core_axis_name="c", subcore_axis_name="s"),
    compiler_params=pltpu.CompilerParams(use_tc_tiling_on_sc=False),
    out_type=jax.ShapeDtypeStruct((N_PAD, C), jnp.float32),
    scratch_types=[
        pltpu.VMEM((G * K,), jnp.int32),
        pltpu.VMEM((G * K, C), jnp.float32),
        pltpu.VMEM((G, C), jnp.float32),
        pltpu.SemaphoreType.DMA,
    ],
)
def _sc_gather(p_ref, i_ref, o_ref, idx_v, rows_v, outb_v, sem):
    wid = lax.axis_index("s") * 2 + lax.axis_index("c")
    grp0 = wid * GPW

    def body(j, carry):
        g = grp0 + j
        pltpu.sync_copy(i_ref.at[g], idx_v)
        pltpu.async_copy(p_ref.at[idx_v], rows_v, sem).wait()
        for gg in range(G):
            for cc in range(C // 16):
                acc = rows_v[gg * K, pl.ds(cc * 16, 16)]
                for k in range(1, K):
                    acc = acc + rows_v[gg * K + k, pl.ds(cc * 16, 16)]
                outb_v[gg, pl.ds(cc * 16, 16)] = acc
        pltpu.sync_copy(outb_v, o_ref.at[pl.ds(g * G, G)])
        return carry

    lax.fori_loop(0, GPW, body, 0)


def _bn_coeffs(s, q, gamma, beta):
    mean = s[0] / N
    var = q[0] / N - mean * mean
    inv = gamma / jnp.sqrt(var + 1e-4)
    a = inv
    b = beta - mean * inv
    return (jnp.broadcast_to(a[None, :], (8, C)),
            jnp.broadcast_to(b[None, :], (8, C)))


def kernel(features, neighbor_idx, neighbor_mask, W1, gamma1, beta1,
           W2, gamma2, beta2):
    feat = jnp.pad(features, ((0, N_PAD - N), (0, 0)))
    idx = neighbor_idx.astype(jnp.int32)
    tgt = jnp.where(neighbor_mask, idx, N) * K + \
        jnp.arange(K, dtype=jnp.int32)[None, :]
    tgt = jnp.pad(tgt, ((0, N_PAD - N), (0, 0)), constant_values=N * K)
    i2d = tgt.reshape(-1, G * K)
    w1f = jnp.transpose(W1, (1, 0, 2)).reshape(C, KC)
    w2f = jnp.transpose(W2, (1, 0, 2)).reshape(C, KC)

    p1 = _matmul(feat, w1f)
    h1 = _sc_gather(p1.reshape(N_PAD * K, C), i2d)
    s1, q1 = _stats(h1)
    a1, b1 = _bn_coeffs(s1, q1, gamma1, beta1)
    p2 = _norm_matmul(h1, a1, b1, w2f)
    h2 = _sc_gather(p2.reshape(N_PAD * K, C), i2d)
    s2, q2 = _stats(h2)
    a2, b2 = _bn_coeffs(s2, q2, gamma2, beta2)
    return _finalize(h2, a2, b2)


# speedup vs baseline: 17.2705x; 1.0138x over previous
import functools

import jax
import jax.numpy as jnp
from jax import lax
from jax.experimental import pallas as pl
from jax.experimental.pallas import tpu as pltpu
from jax.experimental.pallas import tpu_sc as plsc

N = 50000
C = 64
K = 27
KC = K * C
NW = 32
G = 4
GPW = 392
RPW = G * GPW
N_PAD = NW * RPW
MM_BLK = 512
FIN_BLK = 400


def _mm_kernel(x_ref, w_ref, o_ref):
    o_ref[...] = jnp.dot(x_ref[...], w_ref[...],
                         preferred_element_type=jnp.float32)


def _matmul(x, w):
    return pl.pallas_call(
        _mm_kernel,
        grid=(N_PAD // MM_BLK,),
        in_specs=[pl.BlockSpec((MM_BLK, C), lambda i: (i, 0)),
                  pl.BlockSpec((C, KC), lambda i: (0, 0))],
        out_specs=pl.BlockSpec((MM_BLK, KC), lambda i: (i, 0)),
        out_shape=jax.ShapeDtypeStruct((N_PAD, KC), jnp.float32),
    )(x, w)


def _norm_mm_kernel(x_ref, a_ref, b_ref, w_ref, o_ref):
    i = pl.program_id(0)
    x = x_ref[...]
    xn = x * a_ref[0:1, :] + b_ref[0:1, :]
    xn = jnp.where(xn >= 0, xn, 0.05 * xn)
    rows = i * MM_BLK + lax.broadcasted_iota(jnp.int32, (MM_BLK, 1), 0)
    xn = jnp.where(rows < N, xn, 0.0)
    o_ref[...] = jnp.dot(xn, w_ref[...], preferred_element_type=jnp.float32)


def _norm_matmul(x, a, b, w):
    return pl.pallas_call(
        _norm_mm_kernel,
        grid=(N_PAD // MM_BLK,),
        in_specs=[pl.BlockSpec((MM_BLK, C), lambda i: (i, 0)),
                  pl.BlockSpec((8, C), lambda i: (0, 0)),
                  pl.BlockSpec((8, C), lambda i: (0, 0)),
                  pl.BlockSpec((C, KC), lambda i: (0, 0))],
        out_specs=pl.BlockSpec((MM_BLK, KC), lambda i: (i, 0)),
        out_shape=jax.ShapeDtypeStruct((N_PAD, KC), jnp.float32),
    )(x, a, b, w)


def _stats_kernel(x_ref, s_ref, q_ref):
    i = pl.program_id(0)
    x = x_ref[...]
    s = jnp.broadcast_to(jnp.sum(x, axis=0, keepdims=True), (8, C))
    q = jnp.broadcast_to(jnp.sum(x * x, axis=0, keepdims=True), (8, C))

    @pl.when(i == 0)
    def _():
        s_ref[...] = s
        q_ref[...] = q

    @pl.when(i > 0)
    def _():
        s_ref[...] += s
        q_ref[...] += q


def _stats(x):
    return pl.pallas_call(
        _stats_kernel,
        grid=(N_PAD // MM_BLK,),
        in_specs=[pl.BlockSpec((MM_BLK, C), lambda i: (i, 0))],
        out_specs=[pl.BlockSpec((8, C), lambda i: (0, 0)),
                   pl.BlockSpec((8, C), lambda i: (0, 0))],
        out_shape=[jax.ShapeDtypeStruct((8, C), jnp.float32),
                   jax.ShapeDtypeStruct((8, C), jnp.float32)],
    )(x)


def _final_kernel(x_ref, a_ref, b_ref, o_ref):
    xn = x_ref[...] * a_ref[0:1, :] + b_ref[0:1, :]
    o_ref[...] = jnp.where(xn >= 0, xn, 0.05 * xn)


def _finalize(x, a, b):
    return pl.pallas_call(
        _final_kernel,
        grid=(N // FIN_BLK,),
        in_specs=[pl.BlockSpec((FIN_BLK, C), lambda i: (i, 0)),
                  pl.BlockSpec((8, C), lambda i: (0, 0)),
                  pl.BlockSpec((8, C), lambda i: (0, 0))],
        out_specs=pl.BlockSpec((FIN_BLK, C), lambda i: (i, 0)),
        out_shape=jax.ShapeDtypeStruct((N, C), jnp.float32),
    )(x, a, b)


NB = 4
ROUNDS = GPW // NB
RG = NB * G


@functools.partial(
    pl.kernel,
    mesh=plsc.VectorSubcoreMesh(core_axis_name="c", subcore_axis_name="s"),
    compiler_params=pltpu.CompilerParams(use_tc_tiling_on_sc=False),
    out_type=jax.ShapeDtypeStruct((N_PAD, C), jnp.float32),
    scratch_types=[
        pltpu.VMEM((GPW, G * K), jnp.int32),
        pltpu.VMEM((NB, G * K, C), jnp.float32),
        pltpu.VMEM((RG, C), jnp.float32),
        pltpu.SemaphoreType.DMA,
        pltpu.SemaphoreType.DMA,
        pltpu.SemaphoreType.DMA,
        pltpu.SemaphoreType.DMA,
        pltpu.SemaphoreType.DMA,
    ],
)
def _sc_gather(p_ref, i_ref, o_ref, idx_all, rows_v, outb_v,
               sem0, sem1, sem2, sem3, osem):
    sems = (sem0, sem1, sem2, sem3)
    wid = lax.axis_index("s") * 2 + lax.axis_index("c")
    pltpu.sync_copy(i_ref.at[pl.ds(wid * GPW, GPW)], idx_all)
    row0 = wid * RPW

    for b in range(NB):
        pltpu.async_copy(p_ref.at[idx_all.at[b]], rows_v.at[b], sems[b])

    def round_body(r, carry):
        @pl.when(r > 0)
        def _():
            pltpu.make_async_copy(
                outb_v, o_ref.at[pl.ds(row0, RG)], osem).wait()
        for b in range(NB):
            g = r * NB + b
            pltpu.make_async_copy(
                p_ref.at[idx_all.at[b]], rows_v.at[b], sems[b]).wait()
            for gg in range(G):
                for cc in range(C // 16):
                    acc = rows_v[b, gg * K, pl.ds(cc * 16, 16)]
                    for k in range(1, K):
                        acc = acc + rows_v[b, gg * K + k, pl.ds(cc * 16, 16)]
                    outb_v[b * G + gg, pl.ds(cc * 16, 16)] = acc

            @pl.when(r < ROUNDS - 1)
            def _():
                pltpu.async_copy(
                    p_ref.at[idx_all.at[g + NB]], rows_v.at[b], sems[b])
        pltpu.async_copy(
            outb_v, o_ref.at[pl.ds(row0 + r * RG, RG)], osem)
        return carry

    lax.fori_loop(0, ROUNDS, round_body, 0)
    pltpu.make_async_copy(outb_v, o_ref.at[pl.ds(row0, RG)], osem).wait()


def _bn_coeffs(s, q, gamma, beta):
    mean = s[0] / N
    var = q[0] / N - mean * mean
    inv = gamma / jnp.sqrt(var + 1e-4)
    a = inv
    b = beta - mean * inv
    return (jnp.broadcast_to(a[None, :], (8, C)),
            jnp.broadcast_to(b[None, :], (8, C)))


def kernel(features, neighbor_idx, neighbor_mask, W1, gamma1, beta1,
           W2, gamma2, beta2):
    feat = jnp.pad(features, ((0, N_PAD - N), (0, 0)))
    idx = neighbor_idx.astype(jnp.int32)
    tgt = jnp.where(neighbor_mask, idx, N) * K + \
        jnp.arange(K, dtype=jnp.int32)[None, :]
    tgt = jnp.pad(tgt, ((0, N_PAD - N), (0, 0)), constant_values=N * K)
    i2d = tgt.reshape(-1, G * K)
    w1f = jnp.transpose(W1, (1, 0, 2)).reshape(C, KC)
    w2f = jnp.transpose(W2, (1, 0, 2)).reshape(C, KC)

    p1 = _matmul(feat, w1f)
    h1 = _sc_gather(p1.reshape(N_PAD * K, C), i2d)
    s1, q1 = _stats(h1)
    a1, b1 = _bn_coeffs(s1, q1, gamma1, beta1)
    p2 = _norm_matmul(h1, a1, b1, w2f)
    h2 = _sc_gather(p2.reshape(N_PAD * K, C), i2d)
    s2, q2 = _stats(h2)
    a2, b2 = _bn_coeffs(s2, q2, gamma2, beta2)
    return _finalize(h2, a2, b2)
